# baseline (speedup 1.0000x reference)
import functools

import jax
import jax.numpy as jnp
from jax import lax
from jax.experimental import pallas as pl
from jax.experimental.pallas import tpu as pltpu
from jax.experimental.pallas import tpu_sc as plsc

E = 8
TOP_K = 2
DIM = 1024
HIDDEN = 2816
S = 2048
R = S * TOP_K
BLK = 128
R_PAD = R + E * BLK
W_MAX = R_PAD // BLK


def _gate_body(x_ref, wg_ref, idx_ref, w_ref):
    x = x_ref[...]
    wg = wg_ref[...]
    s = lax.dot_general(x, wg, (((1,), (1,)), ((), ())),
                        preferred_element_type=jnp.float32)
    col = lax.broadcasted_iota(jnp.int32, s.shape, 1)
    m1 = jnp.max(s, axis=1, keepdims=True)
    i1 = jnp.min(jnp.where(s == m1, col, E), axis=1, keepdims=True)
    s2 = jnp.where(col == i1, -jnp.inf, s)
    m2 = jnp.max(s2, axis=1, keepdims=True)
    i2 = jnp.min(jnp.where(s2 == m2, col, E), axis=1, keepdims=True)
    e2 = jnp.exp(m2 - m1)
    w1 = 1.0 / (1.0 + e2)
    w2 = e2 / (1.0 + e2)
    idx_ref[...] = jnp.concatenate([i1, i2], axis=1)
    w_ref[...] = jnp.concatenate([w1, w2], axis=1)


def _gate(xf, Wg):
    return pl.pallas_call(
        _gate_body,
        out_shape=(
            jax.ShapeDtypeStruct((S, TOP_K), jnp.int32),
            jax.ShapeDtypeStruct((S, TOP_K), jnp.float32),
        ),
    )(xf, Wg)


def _sc_gather(table, idx, n_rows):
    info = plsc.get_sparse_core_info()
    nw = info.num_cores * info.num_subcores
    b_per_w = n_rows // nw
    chunk = b_per_w
    while chunk * DIM * 4 > 240 * 1024:
        chunk //= 2
    n_chunks = b_per_w // chunk
    mesh = plsc.VectorSubcoreMesh(core_axis_name="c", subcore_axis_name="s")

    @functools.partial(
        pl.kernel, mesh=mesh,
        out_type=jax.ShapeDtypeStruct((n_rows, DIM), jnp.float32),
        scratch_types=[
            pltpu.VMEM((b_per_w,), jnp.int32),
            pltpu.VMEM((chunk, DIM), jnp.float32),
            pltpu.SemaphoreType.DMA,
        ],
    )
    def k(table_hbm, idx_hbm, out_hbm, idx_v, rows_v, sem):
        wid = lax.axis_index("s") * info.num_cores + lax.axis_index("c")
        base = wid * b_per_w
        pltpu.sync_copy(idx_hbm.at[pl.ds(base, b_per_w)], idx_v)
        for c in range(n_chunks):
            pltpu.async_copy(
                table_hbm.at[idx_v.at[pl.ds(c * chunk, chunk)]], rows_v, sem
            ).wait()
            pltpu.sync_copy(rows_v, out_hbm.at[pl.ds(base + c * chunk, chunk)])

    return k(table, idx)


def _sc_combine(rows, pos0, pos1):
    info = plsc.get_sparse_core_info()
    nw = info.num_cores * info.num_subcores
    t_per_w = S // nw
    chunk = 32
    n_chunks = t_per_w // chunk
    vregs = chunk * DIM // 16
    mesh = plsc.VectorSubcoreMesh(core_axis_name="c", subcore_axis_name="s")

    @functools.partial(
        pl.kernel, mesh=mesh,
        out_type=jax.ShapeDtypeStruct((S, DIM), jnp.float32),
        scratch_types=[
            pltpu.VMEM((chunk,), jnp.int32),
            pltpu.VMEM((chunk,), jnp.int32),
            pltpu.VMEM((chunk, DIM), jnp.float32),
            pltpu.VMEM((chunk, DIM), jnp.float32),
            pltpu.SemaphoreType.DMA,
            pltpu.SemaphoreType.DMA,
        ],
    )
    def k(rows_hbm, p0_hbm, p1_hbm, y_hbm, p0_v, p1_v, a_v, b_v, sem0, sem1):
        wid = lax.axis_index("s") * info.num_cores + lax.axis_index("c")
        for c in range(n_chunks):
            base = wid * t_per_w + c * chunk
            pltpu.sync_copy(p0_hbm.at[pl.ds(base, chunk)], p0_v)
            pltpu.sync_copy(p1_hbm.at[pl.ds(base, chunk)], p1_v)
            cp0 = pltpu.async_copy(rows_hbm.at[p0_v], a_v, sem0)
            cp1 = pltpu.async_copy(rows_hbm.at[p1_v], b_v, sem1)
            cp0.wait()
            cp1.wait()

            def body(i, _):
                t = i // (DIM // 16)
                sl = pl.ds((i % (DIM // 16)) * 16, 16)
                a_v[t, sl] = a_v[t, sl] + b_v[t, sl]
                return 0

            lax.fori_loop(0, vregs, body, 0)
            pltpu.sync_copy(a_v, y_hbm.at[pl.ds(base, chunk)])

    return k(rows, pos0, pos1)


def _ffn_body(we_ref, used_ref, xs_ref, w1_ref, w3_ref, w2_ref, wsc_ref,
              out_ref):
    i = pl.program_id(0)

    @pl.when(i < used_ref[0])
    def _():
        x = xs_ref[...].astype(jnp.bfloat16)
        w1 = w1_ref[0]
        w3 = w3_ref[0]
        w2 = w2_ref[0]
        h1 = lax.dot_general(x, w1, (((1,), (1,)), ((), ())),
                             preferred_element_type=jnp.float32)
        h3 = lax.dot_general(x, w3, (((1,), (1,)), ((), ())),
                             preferred_element_type=jnp.float32)
        h = (h1 * jax.nn.sigmoid(h1)) * h3
        y = lax.dot_general(h.astype(jnp.bfloat16), w2,
                            (((1,), (1,)), ((), ())),
                            preferred_element_type=jnp.float32)
        out_ref[...] = y * wsc_ref[...]


def _ffn(we, used, xs, W1b, W3b, W2b, w_pad):
    grid_spec = pltpu.PrefetchScalarGridSpec(
        num_scalar_prefetch=2,
        grid=(W_MAX,),
        in_specs=[
            pl.BlockSpec((BLK, DIM), lambda i, we, u: (i, 0)),
            pl.BlockSpec((1, HIDDEN, DIM), lambda i, we, u: (we[i], 0, 0)),
            pl.BlockSpec((1, HIDDEN, DIM), lambda i, we, u: (we[i], 0, 0)),
            pl.BlockSpec((1, DIM, HIDDEN), lambda i, we, u: (we[i], 0, 0)),
            pl.BlockSpec((BLK, 1), lambda i, we, u: (i, 0)),
        ],
        out_specs=pl.BlockSpec((BLK, DIM), lambda i, we, u: (i, 0)),
    )
    return pl.pallas_call(
        _ffn_body,
        grid_spec=grid_spec,
        out_shape=jax.ShapeDtypeStruct((R_PAD, DIM), jnp.float32),
        compiler_params=pltpu.CompilerParams(
            dimension_semantics=("arbitrary",)),
    )(we, used, xs, W1b, W3b, W2b, w_pad)


@jax.jit
def kernel(x, Wg, W1, W2, W3):
    orig_shape = x.shape
    xf = x.reshape(-1, DIM)

    idx, w = _gate(xf, Wg)

    e_flat = idx.reshape(-1)
    onehot = (e_flat[None, :] == jnp.arange(E, dtype=jnp.int32)[:, None])
    onehot = onehot.astype(jnp.int32)
    csum = jnp.cumsum(onehot, axis=1)
    counts = csum[:, -1]
    rank = jnp.sum(onehot * (csum - 1), axis=0)
    blocks_e = (counts + BLK - 1) // BLK
    pad_counts = blocks_e * BLK
    starts_pad = jnp.cumsum(pad_counts) - pad_counts
    used = jnp.sum(blocks_e).astype(jnp.int32).reshape(1)
    pos_pad = (jnp.sum(onehot * starts_pad[:, None], axis=0) + rank
               ).astype(jnp.int32)
    start_blk = starts_pad // BLK
    blk_ids = jnp.arange(W_MAX, dtype=jnp.int32)
    we = (jnp.sum((blk_ids[:, None] >= start_blk[None, :]).astype(jnp.int32),
                  axis=1) - 1).astype(jnp.int32)

    tok = (jnp.arange(R, dtype=jnp.int32) // TOP_K)
    tok_pad = jnp.zeros((R_PAD,), jnp.int32).at[pos_pad].set(tok)
    w_pad = jnp.zeros((R_PAD,), jnp.float32).at[pos_pad].set(w.reshape(-1))
    w_pad = w_pad.reshape(R_PAD, 1)
    pos0 = pos_pad[0::2]
    pos1 = pos_pad[1::2]

    xs = _sc_gather(xf, tok_pad, R_PAD)

    rows = _ffn(we, used, xs,
                W1.astype(jnp.bfloat16), W3.astype(jnp.bfloat16),
                W2.astype(jnp.bfloat16), w_pad)

    y = _sc_combine(rows, pos0, pos1)
    return y.reshape(orig_shape)


# speedup vs baseline: 1.1266x; 1.1266x over previous
import functools

import jax
import jax.numpy as jnp
from jax import lax
from jax.experimental import pallas as pl
from jax.experimental.pallas import tpu as pltpu
from jax.experimental.pallas import tpu_sc as plsc

E = 8
TOP_K = 2
DIM = 1024
HIDDEN = 2816
S = 2048
R = S * TOP_K
BLK = 256
R_PAD = R + E * BLK
W_MAX = R_PAD // BLK


def _gate_body(x_ref, wg_ref, idx_ref, w_ref):
    x = x_ref[...]
    wg = wg_ref[...]
    s = lax.dot_general(x, wg, (((1,), (1,)), ((), ())),
                        preferred_element_type=jnp.float32)
    col = lax.broadcasted_iota(jnp.int32, s.shape, 1)
    m1 = jnp.max(s, axis=1, keepdims=True)
    i1 = jnp.min(jnp.where(s == m1, col, E), axis=1, keepdims=True)
    s2 = jnp.where(col == i1, -jnp.inf, s)
    m2 = jnp.max(s2, axis=1, keepdims=True)
    i2 = jnp.min(jnp.where(s2 == m2, col, E), axis=1, keepdims=True)
    e2 = jnp.exp(m2 - m1)
    w1 = 1.0 / (1.0 + e2)
    w2 = e2 / (1.0 + e2)
    idx_ref[...] = jnp.concatenate([i1, i2], axis=1)
    w_ref[...] = jnp.concatenate([w1, w2], axis=1)


def _gate(xf, Wg):
    return pl.pallas_call(
        _gate_body,
        out_shape=(
            jax.ShapeDtypeStruct((S, TOP_K), jnp.int32),
            jax.ShapeDtypeStruct((S, TOP_K), jnp.float32),
        ),
    )(xf, Wg)


def _sc_gather(table, idx, n_rows):
    info = plsc.get_sparse_core_info()
    nw = info.num_cores * info.num_subcores
    b_per_w = n_rows // nw
    chunk = b_per_w
    while chunk * DIM * 4 > 240 * 1024:
        chunk //= 2
    n_chunks = b_per_w // chunk
    mesh = plsc.VectorSubcoreMesh(core_axis_name="c", subcore_axis_name="s")

    @functools.partial(
        pl.kernel, mesh=mesh,
        out_type=jax.ShapeDtypeStruct((n_rows, DIM), jnp.float32),
        scratch_types=[
            pltpu.VMEM((b_per_w,), jnp.int32),
            pltpu.VMEM((chunk, DIM), jnp.float32),
            pltpu.SemaphoreType.DMA,
        ],
    )
    def k(table_hbm, idx_hbm, out_hbm, idx_v, rows_v, sem):
        wid = lax.axis_index("s") * info.num_cores + lax.axis_index("c")
        base = wid * b_per_w
        pltpu.sync_copy(idx_hbm.at[pl.ds(base, b_per_w)], idx_v)
        for c in range(n_chunks):
            pltpu.async_copy(
                table_hbm.at[idx_v.at[pl.ds(c * chunk, chunk)]], rows_v, sem
            ).wait()
            pltpu.sync_copy(rows_v, out_hbm.at[pl.ds(base + c * chunk, chunk)])

    return k(table, idx)


def _sc_combine(rows, pos0, pos1):
    info = plsc.get_sparse_core_info()
    nw = info.num_cores * info.num_subcores
    t_per_w = S // nw
    chunk = 32
    n_chunks = t_per_w // chunk
    vregs = chunk * DIM // 16
    mesh = plsc.VectorSubcoreMesh(core_axis_name="c", subcore_axis_name="s")

    @functools.partial(
        pl.kernel, mesh=mesh,
        out_type=jax.ShapeDtypeStruct((S, DIM), jnp.float32),
        scratch_types=[
            pltpu.VMEM((chunk,), jnp.int32),
            pltpu.VMEM((chunk,), jnp.int32),
            pltpu.VMEM((chunk, DIM), jnp.float32),
            pltpu.VMEM((chunk, DIM), jnp.float32),
            pltpu.SemaphoreType.DMA,
            pltpu.SemaphoreType.DMA,
        ],
    )
    def k(rows_hbm, p0_hbm, p1_hbm, y_hbm, p0_v, p1_v, a_v, b_v, sem0, sem1):
        wid = lax.axis_index("s") * info.num_cores + lax.axis_index("c")
        for c in range(n_chunks):
            base = wid * t_per_w + c * chunk
            pltpu.sync_copy(p0_hbm.at[pl.ds(base, chunk)], p0_v)
            pltpu.sync_copy(p1_hbm.at[pl.ds(base, chunk)], p1_v)
            cp0 = pltpu.async_copy(rows_hbm.at[p0_v], a_v, sem0)
            cp1 = pltpu.async_copy(rows_hbm.at[p1_v], b_v, sem1)
            cp0.wait()
            cp1.wait()

            def body(i, _):
                t = i // (DIM // 16)
                sl = pl.ds((i % (DIM // 16)) * 16, 16)
                a_v[t, sl] = a_v[t, sl] + b_v[t, sl]
                return 0

            lax.fori_loop(0, vregs, body, 0)
            pltpu.sync_copy(a_v, y_hbm.at[pl.ds(base, chunk)])

    return k(rows, pos0, pos1)


def _ffn_body(we_ref, used_ref, xs_ref, w1_ref, w3_ref, w2_ref, wsc_ref,
              out_ref):
    i = pl.program_id(0)

    @pl.when(i < used_ref[0])
    def _():
        x = xs_ref[...].astype(jnp.bfloat16)
        w1 = w1_ref[0]
        w3 = w3_ref[0]
        w2 = w2_ref[0]
        h1 = lax.dot_general(x, w1, (((1,), (1,)), ((), ())),
                             preferred_element_type=jnp.float32)
        h3 = lax.dot_general(x, w3, (((1,), (1,)), ((), ())),
                             preferred_element_type=jnp.float32)
        h = (h1 * jax.nn.sigmoid(h1)) * h3
        y = lax.dot_general(h.astype(jnp.bfloat16), w2,
                            (((1,), (1,)), ((), ())),
                            preferred_element_type=jnp.float32)
        out_ref[...] = y * wsc_ref[...]


def _ffn(we, used, xs, W1b, W3b, W2b, w_pad):
    grid_spec = pltpu.PrefetchScalarGridSpec(
        num_scalar_prefetch=2,
        grid=(W_MAX,),
        in_specs=[
            pl.BlockSpec((BLK, DIM), lambda i, we, u: (i, 0)),
            pl.BlockSpec((1, HIDDEN, DIM), lambda i, we, u: (we[i], 0, 0)),
            pl.BlockSpec((1, HIDDEN, DIM), lambda i, we, u: (we[i], 0, 0)),
            pl.BlockSpec((1, DIM, HIDDEN), lambda i, we, u: (we[i], 0, 0)),
            pl.BlockSpec((BLK, 1), lambda i, we, u: (i, 0)),
        ],
        out_specs=pl.BlockSpec((BLK, DIM), lambda i, we, u: (i, 0)),
    )
    return pl.pallas_call(
        _ffn_body,
        grid_spec=grid_spec,
        out_shape=jax.ShapeDtypeStruct((R_PAD, DIM), jnp.float32),
        compiler_params=pltpu.CompilerParams(
            dimension_semantics=("arbitrary",)),
    )(we, used, xs, W1b, W3b, W2b, w_pad)


@jax.jit
def kernel(x, Wg, W1, W2, W3):
    orig_shape = x.shape
    xf = x.reshape(-1, DIM)

    idx, w = _gate(xf, Wg)

    e_flat = idx.reshape(-1)
    onehot = (e_flat[None, :] == jnp.arange(E, dtype=jnp.int32)[:, None])
    onehot = onehot.astype(jnp.int32)
    csum = jnp.cumsum(onehot, axis=1)
    counts = csum[:, -1]
    rank = jnp.sum(onehot * (csum - 1), axis=0)
    blocks_e = (counts + BLK - 1) // BLK
    pad_counts = blocks_e * BLK
    starts_pad = jnp.cumsum(pad_counts) - pad_counts
    used = jnp.sum(blocks_e).astype(jnp.int32).reshape(1)
    pos_pad = (jnp.sum(onehot * starts_pad[:, None], axis=0) + rank
               ).astype(jnp.int32)
    start_blk = starts_pad // BLK
    blk_ids = jnp.arange(W_MAX, dtype=jnp.int32)
    we = (jnp.sum((blk_ids[:, None] >= start_blk[None, :]).astype(jnp.int32),
                  axis=1) - 1).astype(jnp.int32)

    tok = (jnp.arange(R, dtype=jnp.int32) // TOP_K)
    tok_pad = jnp.zeros((R_PAD,), jnp.int32).at[pos_pad].set(tok)
    w_pad = jnp.zeros((R_PAD,), jnp.float32).at[pos_pad].set(w.reshape(-1))
    w_pad = w_pad.reshape(R_PAD, 1)
    pos0 = pos_pad[0::2]
    pos1 = pos_pad[1::2]

    xs = _sc_gather(xf, tok_pad, R_PAD)

    rows = _ffn(we, used, xs,
                W1.astype(jnp.bfloat16), W3.astype(jnp.bfloat16),
                W2.astype(jnp.bfloat16), w_pad)

    y = _sc_combine(rows, pos0, pos1)
    return y.reshape(orig_shape)


# speedup vs baseline: 1.5140x; 1.3439x over previous
import functools

import jax
import jax.numpy as jnp
from jax import lax
from jax.experimental import pallas as pl
from jax.experimental.pallas import tpu as pltpu
from jax.experimental.pallas import tpu_sc as plsc

E = 8
TOP_K = 2
DIM = 1024
HIDDEN = 2816
S = 2048
R = S * TOP_K
BLK = 256
R_PAD = R + E * BLK
W_MAX = R_PAD // BLK


def _gate_body(x_ref, wg_ref, idx_ref, w_ref):
    x = x_ref[...]
    wg = wg_ref[...]
    s = lax.dot_general(x, wg, (((1,), (1,)), ((), ())),
                        preferred_element_type=jnp.float32)
    col = lax.broadcasted_iota(jnp.int32, s.shape, 1)
    m1 = jnp.max(s, axis=1, keepdims=True)
    i1 = jnp.min(jnp.where(s == m1, col, E), axis=1, keepdims=True)
    s2 = jnp.where(col == i1, -jnp.inf, s)
    m2 = jnp.max(s2, axis=1, keepdims=True)
    i2 = jnp.min(jnp.where(s2 == m2, col, E), axis=1, keepdims=True)
    e2 = jnp.exp(m2 - m1)
    w1 = 1.0 / (1.0 + e2)
    w2 = e2 / (1.0 + e2)
    idx_ref[...] = jnp.concatenate([i1, i2], axis=1)
    w_ref[...] = jnp.concatenate([w1, w2], axis=1)


def _gate(xf, Wg):
    return pl.pallas_call(
        _gate_body,
        out_shape=(
            jax.ShapeDtypeStruct((S, TOP_K), jnp.int32),
            jax.ShapeDtypeStruct((S, TOP_K), jnp.float32),
        ),
    )(xf, Wg)


def _sc_dispatch(xf, pos0, pos1):
    info = plsc.get_sparse_core_info()
    nw = info.num_cores * info.num_subcores
    t_per_w = S // nw
    chunk = 32
    n_chunks = t_per_w // chunk
    mesh = plsc.VectorSubcoreMesh(core_axis_name="c", subcore_axis_name="s")

    @functools.partial(
        pl.kernel, mesh=mesh,
        out_type=jax.ShapeDtypeStruct((R_PAD, DIM), jnp.float32),
        scratch_types=[
            pltpu.VMEM((chunk,), jnp.int32),
            pltpu.VMEM((chunk,), jnp.int32),
            pltpu.VMEM((chunk, DIM), jnp.float32),
            pltpu.SemaphoreType.DMA,
            pltpu.SemaphoreType.DMA,
        ],
    )
    def k(xf_hbm, p0_hbm, p1_hbm, out_hbm, p0_v, p1_v, rows_v, sem0, sem1):
        wid = lax.axis_index("s") * info.num_cores + lax.axis_index("c")
        for c in range(n_chunks):
            base = wid * t_per_w + c * chunk
            pltpu.sync_copy(p0_hbm.at[pl.ds(base, chunk)], p0_v)
            pltpu.sync_copy(p1_hbm.at[pl.ds(base, chunk)], p1_v)
            pltpu.sync_copy(xf_hbm.at[pl.ds(base, chunk)], rows_v)
            cp0 = pltpu.async_copy(rows_v, out_hbm.at[p0_v], sem0)
            cp1 = pltpu.async_copy(rows_v, out_hbm.at[p1_v], sem1)
            cp0.wait()
            cp1.wait()

    return k(xf, pos0, pos1)


def _sc_combine(rows, pos0, pos1, w0, w1):
    info = plsc.get_sparse_core_info()
    nw = info.num_cores * info.num_subcores
    t_per_w = S // nw
    chunk = 32
    n_chunks = t_per_w // chunk
    mesh = plsc.VectorSubcoreMesh(core_axis_name="c", subcore_axis_name="s")

    @functools.partial(
        pl.kernel, mesh=mesh,
        out_type=jax.ShapeDtypeStruct((S, DIM), jnp.float32),
        scratch_types=[
            pltpu.VMEM((chunk,), jnp.int32),
            pltpu.VMEM((chunk,), jnp.int32),
            pltpu.VMEM((chunk, 16), jnp.float32),
            pltpu.VMEM((chunk, 16), jnp.float32),
            pltpu.VMEM((chunk, DIM), jnp.float32),
            pltpu.VMEM((chunk, DIM), jnp.float32),
            pltpu.SemaphoreType.DMA,
            pltpu.SemaphoreType.DMA,
        ],
    )
    def k(rows_hbm, p0_hbm, p1_hbm, w0_hbm, w1_hbm, y_hbm,
          p0_v, p1_v, w0_v, w1_v, a_v, b_v, sem0, sem1):
        wid = lax.axis_index("s") * info.num_cores + lax.axis_index("c")
        for c in range(n_chunks):
            base = wid * t_per_w + c * chunk
            pltpu.sync_copy(p0_hbm.at[pl.ds(base, chunk)], p0_v)
            pltpu.sync_copy(p1_hbm.at[pl.ds(base, chunk)], p1_v)
            pltpu.sync_copy(w0_hbm.at[pl.ds(base, chunk)], w0_v)
            pltpu.sync_copy(w1_hbm.at[pl.ds(base, chunk)], w1_v)
            cp0 = pltpu.async_copy(rows_hbm.at[p0_v], a_v, sem0)
            cp1 = pltpu.async_copy(rows_hbm.at[p1_v], b_v, sem1)
            cp0.wait()
            cp1.wait()

            def body(t, _):
                wa = w0_v[t, :]
                wb = w1_v[t, :]

                def inner(j, _):
                    sl = pl.ds(j * 16, 16)
                    a_v[t, sl] = wa * a_v[t, sl] + wb * b_v[t, sl]
                    return 0

                lax.fori_loop(0, DIM // 16, inner, 0)
                return 0

            lax.fori_loop(0, chunk, body, 0)
            pltpu.sync_copy(a_v, y_hbm.at[pl.ds(base, chunk)])

    return k(rows, pos0, pos1, w0, w1)


def _ffn_body(we_ref, used_ref, xs_ref, w1_ref, w3_ref, w2_ref, out_ref):
    i = pl.program_id(0)

    @pl.when(i < used_ref[0])
    def _():
        x = xs_ref[...].astype(jnp.bfloat16)
        w1 = w1_ref[0]
        w3 = w3_ref[0]
        w2 = w2_ref[0]
        h1 = lax.dot_general(x, w1, (((1,), (1,)), ((), ())),
                             preferred_element_type=jnp.float32)
        h3 = lax.dot_general(x, w3, (((1,), (1,)), ((), ())),
                             preferred_element_type=jnp.float32)
        h = (h1 * jax.nn.sigmoid(h1)) * h3
        y = lax.dot_general(h.astype(jnp.bfloat16), w2,
                            (((1,), (1,)), ((), ())),
                            preferred_element_type=jnp.float32)
        out_ref[...] = y


def _ffn(we, used, xs, W1b, W3b, W2b):
    grid_spec = pltpu.PrefetchScalarGridSpec(
        num_scalar_prefetch=2,
        grid=(W_MAX,),
        in_specs=[
            pl.BlockSpec((BLK, DIM), lambda i, we, u: (i, 0)),
            pl.BlockSpec((1, HIDDEN, DIM), lambda i, we, u: (we[i], 0, 0)),
            pl.BlockSpec((1, HIDDEN, DIM), lambda i, we, u: (we[i], 0, 0)),
            pl.BlockSpec((1, DIM, HIDDEN), lambda i, we, u: (we[i], 0, 0)),
        ],
        out_specs=pl.BlockSpec((BLK, DIM), lambda i, we, u: (i, 0)),
    )
    return pl.pallas_call(
        _ffn_body,
        grid_spec=grid_spec,
        out_shape=jax.ShapeDtypeStruct((R_PAD, DIM), jnp.float32),
        compiler_params=pltpu.CompilerParams(
            dimension_semantics=("arbitrary",)),
    )(we, used, xs, W1b, W3b, W2b)


@jax.jit
def kernel(x, Wg, W1, W2, W3):
    orig_shape = x.shape
    xf = x.reshape(-1, DIM)

    idx, w = _gate(xf, Wg)

    e_flat = idx.reshape(-1)
    onehot = (e_flat[None, :] == jnp.arange(E, dtype=jnp.int32)[:, None])
    onehot = onehot.astype(jnp.int32)
    csum = jnp.cumsum(onehot, axis=1)
    counts = csum[:, -1]
    rank = jnp.sum(onehot * (csum - 1), axis=0)
    blocks_e = (counts + BLK - 1) // BLK
    pad_counts = blocks_e * BLK
    starts_pad = jnp.cumsum(pad_counts) - pad_counts
    used = jnp.sum(blocks_e).astype(jnp.int32).reshape(1)
    pos_pad = (jnp.sum(onehot * starts_pad[:, None], axis=0) + rank
               ).astype(jnp.int32)
    start_blk = starts_pad // BLK
    blk_ids = jnp.arange(W_MAX, dtype=jnp.int32)
    we = (jnp.sum((blk_ids[:, None] >= start_blk[None, :]).astype(jnp.int32),
                  axis=1) - 1).astype(jnp.int32)

    pos0 = pos_pad[0::2]
    pos1 = pos_pad[1::2]
    w0 = jnp.broadcast_to(w[:, 0:1], (S, 16))
    w1 = jnp.broadcast_to(w[:, 1:2], (S, 16))

    xs = _sc_dispatch(xf, pos0, pos1)

    rows = _ffn(we, used, xs,
                W1.astype(jnp.bfloat16), W3.astype(jnp.bfloat16),
                W2.astype(jnp.bfloat16))

    y = _sc_combine(rows, pos0, pos1, w0, w1)
    return y.reshape(orig_shape)
